# Initial kernel scaffold; baseline (speedup 1.0000x reference)
import functools

import jax
import jax.numpy as jnp
from jax import lax
from jax.experimental import pallas as pl
from jax.experimental.pallas import tpu as pltpu
from jax.experimental.pallas import tpu_sc as plsc

N = 10000
E = 320000
D_IN = 128
D_H = 32
D_OUT = 64

NC = 2
NS = 16
NW = NC * NS
L = 16

CHUNK = 128
K = -(-E // (NW * CHUNK))
EPAD = NW * K * CHUNK
NPAD = 10240
RPT = NPAD // NS
ZR = 128

_MESH = plsc.VectorSubcoreMesh(
    core_axis_name="c", subcore_axis_name="s", num_cores=NC, num_subcores=NS)


@functools.partial(
    pl.kernel,
    out_type=jax.ShapeDtypeStruct((NC, NPAD), jnp.float32),
    mesh=_MESH,
    scratch_types=[
        pltpu.VMEM((K, CHUNK), jnp.int32),
        pltpu.VMEM((CHUNK,), jnp.float32),
        pltpu.VMEM((RPT,), jnp.float32),
        pltpu.VMEM_SHARED((NPAD,), jnp.float32),
    ],
)
def _deg_kernel(dst_hbm, out_hbm, dst_v, ones_v, zrow_v, acc):
    c = lax.axis_index("c")
    s = lax.axis_index("s")
    wid = c * NS + s
    for i in range(CHUNK // L):
        ones_v[pl.ds(i * L, L)] = jnp.ones((L,), jnp.float32)
    for i in range(RPT // L):
        zrow_v[pl.ds(i * L, L)] = jnp.zeros((L,), jnp.float32)
    pltpu.sync_copy(zrow_v, acc.at[pl.ds(s * RPT, RPT)])
    pltpu.sync_copy(dst_hbm.at[wid], dst_v)
    plsc.subcore_barrier()

    def body(j, carry):
        pltpu.sync_copy(ones_v, acc.at[dst_v.at[j]], add=True)
        return carry

    lax.fori_loop(0, K, body, 0)
    plsc.subcore_barrier()
    pltpu.sync_copy(acc.at[pl.ds(s * RPT, RPT)],
                    out_hbm.at[c, pl.ds(s * RPT, RPT)])


def _make_scatter_kernel(D):
    @functools.partial(
        pl.kernel,
        out_type=jax.ShapeDtypeStruct((NC, NPAD, D), jnp.float32),
        mesh=_MESH,
        scratch_types=[
            pltpu.VMEM((K, CHUNK), jnp.int32),
            pltpu.VMEM((K, CHUNK), jnp.int32),
            pltpu.VMEM((2, CHUNK, D), jnp.float32),
            pltpu.VMEM((ZR, D), jnp.float32),
            pltpu.VMEM_SHARED((NPAD, D), jnp.float32),
            pltpu.SemaphoreType.DMA,
        ],
    )
    def _scatter(g_hbm, src_hbm, dst_hbm, out_hbm,
                 src_v, dst_v, rows_v, zb_v, acc, sem):
        c = lax.axis_index("c")
        s = lax.axis_index("s")
        wid = c * NS + s

        def zfill(i, carry):
            for dj in range(D // L):
                zb_v[i, pl.ds(dj * L, L)] = jnp.zeros((L,), jnp.float32)
            return carry

        lax.fori_loop(0, ZR, zfill, 0)
        for t in range(RPT // ZR):
            pltpu.sync_copy(zb_v, acc.at[pl.ds(s * RPT + t * ZR, ZR)])
        pltpu.sync_copy(src_hbm.at[wid], src_v)
        pltpu.sync_copy(dst_hbm.at[wid], dst_v)
        plsc.subcore_barrier()

        def body(j, carry):
            pltpu.async_copy(g_hbm.at[src_v.at[j]], rows_v.at[0], sem).wait()
            pltpu.sync_copy(rows_v.at[0], acc.at[dst_v.at[j]], add=True)
            return carry

        lax.fori_loop(0, K, body, 0)
        plsc.subcore_barrier()
        pltpu.sync_copy(acc.at[pl.ds(s * RPT, RPT)],
                        out_hbm.at[c, pl.ds(s * RPT, RPT)])

    return _scatter


_scatter32 = _make_scatter_kernel(D_H)
_scatter64 = _make_scatter_kernel(D_OUT)


def _tc_g1_body(x_ref, w1_ref, degpt_ref, g1_ref, dis_ref):
    deg = degpt_ref[:, 0:1] + degpt_ref[:, 1:2] + 1.0
    dis = lax.rsqrt(deg)[:N, :]
    h1 = jnp.dot(x_ref[...], w1_ref[...],
                 preferred_element_type=jnp.float32)
    g1_ref[...] = h1 * dis
    dis_ref[...] = dis


def _tc_mid_body(aggp_ref, g1_ref, dis_ref, b1_ref, w2_ref, g2_ref):
    agg = aggp_ref[0, :N, :] + aggp_ref[1, :N, :]
    dis = dis_ref[...]
    t = dis * (agg + g1_ref[...]) + b1_ref[...]
    t = jnp.where(t > 0, t, jnp.exp(jnp.minimum(t, 0.0)) - 1.0)
    mean = jnp.mean(t, axis=0, keepdims=True)
    var = jnp.mean((t - mean) ** 2, axis=0, keepdims=True)
    t = (t - mean) * lax.rsqrt(var + 1e-5)
    t = jnp.maximum(t, 0.0)
    h2 = jnp.dot(t, w2_ref[...], preferred_element_type=jnp.float32)
    g2_ref[...] = h2 * dis


def _tc_out_body(aggp_ref, g2_ref, dis_ref, b2_ref, out_ref):
    agg = aggp_ref[0, :N, :] + aggp_ref[1, :N, :]
    o = dis_ref[...] * (agg + g2_ref[...]) + b2_ref[...]
    m = jnp.max(o, axis=1, keepdims=True)
    e = jnp.exp(o - m)
    lse = jnp.log(jnp.sum(e, axis=1, keepdims=True)) + m
    out_ref[...] = o - lse


_tc_g1 = pl.pallas_call(
    _tc_g1_body,
    out_shape=(jax.ShapeDtypeStruct((N, D_H), jnp.float32),
               jax.ShapeDtypeStruct((N, 1), jnp.float32)),
)

_tc_mid = pl.pallas_call(
    _tc_mid_body,
    out_shape=jax.ShapeDtypeStruct((N, D_OUT), jnp.float32),
)

_tc_out = pl.pallas_call(
    _tc_out_body,
    out_shape=jax.ShapeDtypeStruct((N, D_OUT), jnp.float32),
)


def kernel(x, edge_index, W1, b1, W2, b2):
    src = edge_index[0]
    dst = edge_index[1]
    pad = EPAD - E
    src_r = jnp.concatenate(
        [src, jnp.zeros((pad,), src.dtype)]).reshape(NW, K, CHUNK)
    dst_r = jnp.concatenate(
        [dst, jnp.full((pad,), N, dst.dtype)]).reshape(NW, K, CHUNK)

    deg_p = _deg_kernel(dst_r)
    g1, dis = _tc_g1(x, W1, deg_p.T)
    agg1 = _scatter32(g1, src_r, dst_r)
    g2 = _tc_mid(agg1, g1, dis, b1, W2)
    agg2 = _scatter64(g2, src_r, dst_r)
    return _tc_out(agg2, g2, dis, b2)


# speedup vs baseline: 23.1253x; 23.1253x over previous
import functools

import jax
import jax.numpy as jnp
from jax import lax
from jax.experimental import pallas as pl
from jax.experimental.pallas import tpu as pltpu
from jax.experimental.pallas import tpu_sc as plsc

N = 10000
E = 320000
D_IN = 128
D_H = 32
D_OUT = 64

NC = 2
NS = 16
NW = NC * NS
L = 16

CHUNK = 128
K = -(-E // (NW * CHUNK))
EPAD = NW * K * CHUNK
NPAD = 10240
RPT = NPAD // NS
ZR = 128

_MESH = plsc.VectorSubcoreMesh(
    core_axis_name="c", subcore_axis_name="s", num_cores=NC, num_subcores=NS)


@functools.partial(
    pl.kernel,
    out_type=jax.ShapeDtypeStruct((NC, NPAD), jnp.float32),
    mesh=_MESH,
    scratch_types=[
        pltpu.VMEM((K, CHUNK), jnp.int32),
        pltpu.VMEM((CHUNK,), jnp.float32),
        pltpu.VMEM((RPT,), jnp.float32),
        pltpu.VMEM_SHARED((NPAD,), jnp.float32),
    ],
)
def _deg_kernel(dst_hbm, out_hbm, dst_v, ones_v, zrow_v, acc):
    c = lax.axis_index("c")
    s = lax.axis_index("s")
    wid = c * NS + s
    for i in range(CHUNK // L):
        ones_v[pl.ds(i * L, L)] = jnp.ones((L,), jnp.float32)
    for i in range(RPT // L):
        zrow_v[pl.ds(i * L, L)] = jnp.zeros((L,), jnp.float32)
    pltpu.sync_copy(zrow_v, acc.at[pl.ds(s * RPT, RPT)])
    pltpu.sync_copy(dst_hbm.at[wid], dst_v)
    plsc.subcore_barrier()

    def body(j, carry):
        pltpu.sync_copy(ones_v, acc.at[dst_v.at[j]], add=True)
        return carry

    lax.fori_loop(0, K, body, 0)
    plsc.subcore_barrier()
    pltpu.sync_copy(acc.at[pl.ds(s * RPT, RPT)],
                    out_hbm.at[c, pl.ds(s * RPT, RPT)])


def _make_scatter_kernel(D):
    @functools.partial(
        pl.kernel,
        out_type=jax.ShapeDtypeStruct((NC, NPAD, D), jnp.float32),
        mesh=_MESH,
        compiler_params=pltpu.CompilerParams(use_tc_tiling_on_sc=False),
        scratch_types=[
            pltpu.VMEM((K, CHUNK), jnp.int32),
            pltpu.VMEM((K, CHUNK), jnp.int32),
            pltpu.VMEM((2, CHUNK, D), jnp.float32),
            pltpu.VMEM((ZR, D), jnp.float32),
            pltpu.VMEM_SHARED((NPAD, D), jnp.float32),
            pltpu.SemaphoreType.DMA,
        ],
    )
    def _scatter(g_hbm, src_hbm, dst_hbm, out_hbm,
                 src_v, dst_v, rows_v, zb_v, acc, sem):
        c = lax.axis_index("c")
        s = lax.axis_index("s")
        wid = c * NS + s

        def zfill(i, carry):
            for dj in range(D // L):
                zb_v[i, pl.ds(dj * L, L)] = jnp.zeros((L,), jnp.float32)
            return carry

        lax.fori_loop(0, ZR, zfill, 0)
        for t in range(RPT // ZR):
            pltpu.sync_copy(zb_v, acc.at[pl.ds(s * RPT + t * ZR, ZR)])
        pltpu.sync_copy(src_hbm.at[wid], src_v)
        pltpu.sync_copy(dst_hbm.at[wid], dst_v)
        plsc.subcore_barrier()

        def body(j, carry):
            pltpu.async_copy(g_hbm.at[src_v.at[j]], rows_v.at[0], sem).wait()
            pltpu.sync_copy(rows_v.at[0], acc.at[dst_v.at[j]], add=True)
            return carry

        lax.fori_loop(0, K, body, 0)
        plsc.subcore_barrier()
        pltpu.sync_copy(acc.at[pl.ds(s * RPT, RPT)],
                        out_hbm.at[c, pl.ds(s * RPT, RPT)])

    return _scatter


_scatter32 = _make_scatter_kernel(D_H)
_scatter64 = _make_scatter_kernel(D_OUT)


def _tc_g1_body(x_ref, w1_ref, degpt_ref, g1_ref, dis_ref):
    deg = degpt_ref[:, 0:1] + degpt_ref[:, 1:2] + 1.0
    dis = lax.rsqrt(deg)[:N, :]
    h1 = jnp.dot(x_ref[...], w1_ref[...],
                 preferred_element_type=jnp.float32)
    g1_ref[...] = h1 * dis
    dis_ref[...] = dis


def _tc_mid_body(aggp_ref, g1_ref, dis_ref, b1_ref, w2_ref, g2_ref):
    agg = aggp_ref[0, :N, :] + aggp_ref[1, :N, :]
    dis = dis_ref[...]
    t = dis * (agg + g1_ref[...]) + b1_ref[...]
    t = jnp.where(t > 0, t, jnp.exp(jnp.minimum(t, 0.0)) - 1.0)
    mean = jnp.mean(t, axis=0, keepdims=True)
    var = jnp.mean((t - mean) ** 2, axis=0, keepdims=True)
    t = (t - mean) * lax.rsqrt(var + 1e-5)
    t = jnp.maximum(t, 0.0)
    h2 = jnp.dot(t, w2_ref[...], preferred_element_type=jnp.float32)
    g2_ref[...] = h2 * dis


def _tc_out_body(aggp_ref, g2_ref, dis_ref, b2_ref, out_ref):
    agg = aggp_ref[0, :N, :] + aggp_ref[1, :N, :]
    o = dis_ref[...] * (agg + g2_ref[...]) + b2_ref[...]
    m = jnp.max(o, axis=1, keepdims=True)
    e = jnp.exp(o - m)
    lse = jnp.log(jnp.sum(e, axis=1, keepdims=True)) + m
    out_ref[...] = o - lse


_tc_g1 = pl.pallas_call(
    _tc_g1_body,
    out_shape=(jax.ShapeDtypeStruct((N, D_H), jnp.float32),
               jax.ShapeDtypeStruct((N, 1), jnp.float32)),
)

_tc_mid = pl.pallas_call(
    _tc_mid_body,
    out_shape=jax.ShapeDtypeStruct((N, D_OUT), jnp.float32),
)

_tc_out = pl.pallas_call(
    _tc_out_body,
    out_shape=jax.ShapeDtypeStruct((N, D_OUT), jnp.float32),
)


def kernel(x, edge_index, W1, b1, W2, b2):
    src = edge_index[0]
    dst = edge_index[1]
    pad = EPAD - E
    src_r = jnp.concatenate(
        [src, jnp.zeros((pad,), src.dtype)]).reshape(NW, K, CHUNK)
    dst_r = jnp.concatenate(
        [dst, jnp.full((pad,), N, dst.dtype)]).reshape(NW, K, CHUNK)

    deg_p = _deg_kernel(dst_r)
    g1, dis = _tc_g1(x, W1, deg_p.T)
    agg1 = _scatter32(g1, src_r, dst_r)
    g2 = _tc_mid(agg1, g1, dis, b1, W2)
    agg2 = _scatter64(g2, src_r, dst_r)
    return _tc_out(agg2, g2, dis, b2)
